# baseline (speedup 1.0000x reference)
import functools
import jax
import jax.numpy as jnp
from jax import lax
from jax.experimental import pallas as pl
from jax.experimental.pallas import tpu as pltpu
from jax.experimental.pallas import tpu_sc as plsc

VOCAB = 32000
ROWS = 1024
NC, NS, L = 2, 16, 16
NW = NC * NS
RPW = ROWS // NW
GROUP = 2
NGRP = RPW // GROUP
BUFW = GROUP * VOCAB
VECS = BUFW // L


def _group_mask_cols(g, hot_lo, hot_hi, lane):
    gp = g % 8
    src = hot_lo if g < 8 else hot_hi
    mask = (lane >= 2 * gp) & (lane < 2 * gp + 2)
    fidx = (lane & 1) * VOCAB + src
    return mask, fidx


def _sc_body(ids_hbm, out_hbm, ids_v, buf0, buf1, sem0, sem1):
    wid = lax.axis_index("s") * NC + lax.axis_index("c")
    base = wid * RPW
    pltpu.sync_copy(ids_hbm.at[pl.ds(base, RPW)], ids_v)

    fill = jnp.full((L,), -20.0, jnp.float32)

    def fill_body(i, _):
        off = i * L
        buf0[pl.ds(off, L)] = fill
        buf1[pl.ds(off, L)] = fill
        return 0

    lax.fori_loop(0, VECS, fill_body, 0)

    lane = lax.iota(jnp.int32, L)
    hot_lo = lax.rem(ids_v[pl.ds(0, L)] + 1, VOCAB)
    hot_hi = lax.rem(ids_v[pl.ds(L, L)] + 1, VOCAB)
    v20 = jnp.full((L,), 20.0, jnp.float32)
    vm20 = jnp.full((L,), -20.0, jnp.float32)
    bufs = (buf0, buf1)
    sems = (sem0, sem1)

    def copy_out(g, slot):
        return pltpu.make_async_copy(
            bufs[slot],
            out_hbm.at[pl.ds((base + g * GROUP) * VOCAB, BUFW)],
            sems[slot],
        )

    for g in range(NGRP):
        slot = g % 2
        if g >= 2:
            copy_out(g - 2, slot).wait()
            pmask, pfidx = _group_mask_cols(g - 2, hot_lo, hot_hi, lane)
            plsc.store_scatter(bufs[slot], [pfidx], vm20, mask=pmask)
        mask, fidx = _group_mask_cols(g, hot_lo, hot_hi, lane)
        plsc.store_scatter(bufs[slot], [fidx], v20, mask=mask)
        copy_out(g, slot).start()

    copy_out(NGRP - 2, (NGRP - 2) % 2).wait()
    copy_out(NGRP - 1, (NGRP - 1) % 2).wait()


_sc_kernel = functools.partial(
    pl.kernel,
    mesh=plsc.VectorSubcoreMesh(core_axis_name="c", subcore_axis_name="s"),
    out_type=jax.ShapeDtypeStruct((ROWS * VOCAB,), jnp.float32),
    scratch_types=[
        pltpu.VMEM((RPW,), jnp.int32),
        pltpu.VMEM((BUFW,), jnp.float32),
        pltpu.VMEM((BUFW,), jnp.float32),
        pltpu.SemaphoreType.DMA,
        pltpu.SemaphoreType.DMA,
    ],
    compiler_params=pltpu.CompilerParams(needs_layout_passes=False),
)(_sc_body)


def kernel(input_ids):
    B, S = input_ids.shape
    ids = input_ids.reshape(B * S).astype(jnp.int32)
    out = _sc_kernel(ids)
    return out.reshape(B, S, VOCAB)
